# baseline (speedup 1.0000x reference)
import functools

import jax
import jax.numpy as jnp
from jax import lax
from jax.experimental import pallas as pl
from jax.experimental.pallas import tpu as pltpu
from jax.experimental.pallas import tpu_sc as plsc

_LOWER = -0.22703196
_UPPER = 0.36853024
_K = 5
_DIM = 3
_KT = 125
_S = 8
_N = 10000
_E = 160000
_EP = 163840
_OP = 16
_KP = 128
_CH = 128
_NCH = _EP // _CH
_NW = 32
_CPW = _NCH // _NW
_MB = 200
_EB = 1280
_NP = 10240
_RPW = _NP // 16
_NG = _N // 8


def _prep_body(ea_ref, ei_ref, fid_ref, bas_ref):
    i = pl.program_id(0)
    gid = i * _EB + lax.broadcasted_iota(jnp.int32, (1, _EB), 1)
    valid = (gid < _E).astype(jnp.float32)
    v = ea_ref[...] * (_K - 1.0)
    bot_f = jnp.floor(v)
    frac = v - bot_f
    bot = bot_f.astype(jnp.int32)
    src = ei_ref[0:1, :]
    dst = ei_ref[1:2, :]
    rows_b, rows_i = [], []
    for s in range(_S):
        w = jnp.ones((1, _EB), jnp.float32)
        idx = jnp.zeros((1, _EB), jnp.int32)
        stride = 1
        for d in range(_DIM):
            off = (s >> d) & 1
            fd = frac[d:d + 1, :]
            wd = fd if off else 1.0 - fd
            idd = jnp.clip(bot[d:d + 1, :] + off, 0, _K - 1)
            w = w * wd
            idx = idx + idd * stride
            stride *= _K
        rows_b.append(w * valid)
        fi = ((src >> 3) * 1024 + (idx >> 3) * 64
              + (src & 7) * 8 + (idx & 7))
        rows_i.append(fi)
    zpad = jnp.zeros((7, _EB), jnp.int32)
    rows16 = jnp.concatenate(rows_i + [dst, zpad], axis=0)
    rows8 = jnp.concatenate(rows_b, axis=0)
    for j in range(_EB // _CH):
        fid_ref[j] = rows16[:, j * _CH:(j + 1) * _CH]
        bas_ref[j] = rows8[:, j * _CH:(j + 1) * _CH]


_prep = pl.pallas_call(
    _prep_body,
    grid=(_EP // _EB,),
    in_specs=[
        pl.BlockSpec((_DIM, _EB), lambda i: (0, i)),
        pl.BlockSpec((2, _EB), lambda i: (0, i)),
    ],
    out_specs=[
        pl.BlockSpec((_EB // _CH, 16, _CH), lambda i: (i, 0, 0)),
        pl.BlockSpec((_EB // _CH, _S, _CH), lambda i: (i, 0, 0)),
    ],
    out_shape=[
        jax.ShapeDtypeStruct((_NCH, 16, _CH), jnp.int32),
        jax.ShapeDtypeStruct((_NCH, _S, _CH), jnp.float32),
    ],
)


def _elu(u):
    return jnp.where(u > 0.0, u, jnp.exp(u) - 1.0)


def _write_z(z_ref, h, wt_ref):
    for c in range(16):
        zc = jnp.dot(h, wt_ref[:, c, :], preferred_element_type=jnp.float32,
                     precision=lax.Precision.HIGHEST)
        z_ref[:, c, :, :] = zc.reshape(_MB // 8, 8, 128)


def _mm1_body(x_ref, wt_ref, h_ref, z_ref):
    t = (x_ref[...] - _LOWER) / (_UPPER - _LOWER) * 20.0 - 10.0
    t = jnp.clip(t, -10.0, 10.0)
    col0 = (lax.broadcasted_iota(jnp.int32, (1, _OP), 1) == 0)
    h = t * col0.astype(jnp.float32)
    h_ref[...] = h
    _write_z(z_ref, h, wt_ref)


_Z4 = jax.ShapeDtypeStruct((_NG, 16, 8, 128), jnp.float32)
_z4_spec = pl.BlockSpec((_MB // 8, 16, 8, 128), lambda i: (i, 0, 0, 0))
_wt_spec = pl.BlockSpec((16, _OP, 128), lambda i: (0, 0, 0))

_mm1 = pl.pallas_call(
    _mm1_body,
    grid=(_N // _MB,),
    in_specs=[
        pl.BlockSpec((_MB, 1), lambda i: (i, 0)),
        _wt_spec,
    ],
    out_specs=[
        pl.BlockSpec((_MB, _OP), lambda i: (i, 0)),
        _z4_spec,
    ],
    out_shape=[
        jax.ShapeDtypeStruct((_N, _OP), jnp.float32),
        _Z4,
    ],
)


def _mm_body(p_ref, hp_ref, root_ref, b_ref, wt_ref, h_ref, z_ref):
    agg = p_ref[0] + p_ref[1]
    r = jnp.dot(hp_ref[...], root_ref[...], preferred_element_type=jnp.float32,
                     precision=lax.Precision.HIGHEST)
    h = _elu(agg + r + b_ref[...])
    h_ref[...] = h
    _write_z(z_ref, h, wt_ref)


_mm = pl.pallas_call(
    _mm_body,
    grid=(_N // _MB,),
    in_specs=[
        pl.BlockSpec((2, _MB, _OP), lambda i: (0, i, 0)),
        pl.BlockSpec((_MB, _OP), lambda i: (i, 0)),
        pl.BlockSpec((_OP, _OP), lambda i: (0, 0)),
        pl.BlockSpec((1, _OP), lambda i: (0, 0)),
        _wt_spec,
    ],
    out_specs=[
        pl.BlockSpec((_MB, _OP), lambda i: (i, 0)),
        _z4_spec,
    ],
    out_shape=[
        jax.ShapeDtypeStruct((_N, _OP), jnp.float32),
        _Z4,
    ],
)


def _fin_body(p_ref, hp_ref, root_ref, b_ref, y_ref):
    agg = p_ref[0] + p_ref[1]
    r = jnp.dot(hp_ref[...], root_ref[...], preferred_element_type=jnp.float32,
                     precision=lax.Precision.HIGHEST)
    h = _elu(agg + r + b_ref[...])
    y_ref[...] = h[:, 0:1]


_fin = pl.pallas_call(
    _fin_body,
    grid=(_N // _MB,),
    in_specs=[
        pl.BlockSpec((2, _MB, _OP), lambda i: (0, i, 0)),
        pl.BlockSpec((_MB, _OP), lambda i: (i, 0)),
        pl.BlockSpec((_OP, _OP), lambda i: (0, 0)),
        pl.BlockSpec((1, _OP), lambda i: (0, 0)),
    ],
    out_specs=pl.BlockSpec((_MB, 1), lambda i: (i, 0)),
    out_shape=jax.ShapeDtypeStruct((_N, 1), jnp.float32),
)


_mesh = plsc.VectorSubcoreMesh(
    core_axis_name="c", subcore_axis_name="s", num_cores=2, num_subcores=16)


@functools.partial(
    pl.kernel,
    out_type=jax.ShapeDtypeStruct((2, _NP, _OP), jnp.float32),
    mesh=_mesh,
    compiler_params=pltpu.CompilerParams(use_tc_tiling_on_sc=False),
    scratch_types=[
        [pltpu.VMEM((2, 16, _CH), jnp.int32)] * 2,
        [pltpu.VMEM((2, _S, _CH), jnp.float32)] * 2,
        [pltpu.VMEM((2, _CH), jnp.int32)] * 2,
        [pltpu.VMEM((2 * _S * _CH, _OP), jnp.float32)] * 2,
        [pltpu.VMEM((2 * _CH, _OP), jnp.float32)] * 2,
        pltpu.VMEM((_CH, _OP), jnp.float32),
        pltpu.VMEM_SHARED((_NP, _OP), jnp.float32),
        [pltpu.SemaphoreType.DMA] * 2,
        [pltpu.SemaphoreType.DMA] * 2,
        [pltpu.SemaphoreType.DMA] * 2,
    ],
)
def _sc_layer(z_hbm, fid_hbm, bas_hbm, p_hbm,
              fid_v, bas_v, dst_v, rows_v, msg_v, zv, agg,
              sem_i, sem_g, sem_s):
    cid = lax.axis_index("c")
    sid = lax.axis_index("s")
    zero16 = jnp.zeros((16,), jnp.float32)
    for i in range(_CH):
        zv[i] = zero16
    for t in range(_RPW // _CH):
        pltpu.sync_copy(zv, agg.at[pl.ds(sid * _RPW + t * _CH, _CH)])
    plsc.subcore_barrier()

    wid = sid * 2 + cid
    base = wid * _CPW

    def load_in(slot, lch):
        ch = base + jnp.minimum(lch, _CPW // 2 - 1) * 2
        pltpu.async_copy(fid_hbm.at[pl.ds(ch, 2)], fid_v[slot], sem_i[slot])
        pltpu.async_copy(bas_hbm.at[pl.ds(ch, 2)], bas_v[slot], sem_i[slot])

    def wait_in(slot):
        pltpu.make_async_copy(fid_hbm.at[pl.ds(0, 2)], fid_v[slot],
                              sem_i[slot]).wait()
        pltpu.make_async_copy(bas_hbm.at[pl.ds(0, 2)], bas_v[slot],
                              sem_i[slot]).wait()

    def fire_gathers(slot):
        for u in range(2):
            for r in range(_S):
                pltpu.async_copy(
                    z_hbm.at[fid_v[slot].at[u, r]],
                    rows_v[slot].at[pl.ds((u * _S + r) * _CH, _CH)],
                    sem_g[slot])

    def wait_gathers(slot):
        for u in range(2):
            for r in range(_S):
                pltpu.make_async_copy(
                    z_hbm.at[fid_v[slot].at[u, r]],
                    rows_v[slot].at[pl.ds((u * _S + r) * _CH, _CH)],
                    sem_g[slot]).wait()

    def wait_scatter(slot):
        for u in range(2):
            pltpu.make_async_copy(msg_v[slot].at[pl.ds(u * _CH, _CH)],
                                  agg.at[dst_v[slot].at[u]],
                                  sem_s[slot]).wait()

    def compute_and_scatter(slot, first):
        rv = rows_v[slot]
        bv = bas_v[slot]
        mv = msg_v[slot]

        @pl.when(jnp.logical_not(first))
        def _():
            wait_scatter(slot)
        for u in range(2):
            for q in range(_CH // 16):
                dst_v[slot][u, pl.ds(q * 16, 16)] = (
                    fid_v[slot][u, 8, pl.ds(q * 16, 16)])
        for u in range(2):
            robase = u * _S * _CH
            mobase = u * _CH

            def group(g, carry):
                base16 = g * 32
                bv1 = [bv[u, r, pl.ds(base16, 16)] for r in range(_S)]
                bv2 = [bv[u, r, pl.ds(base16 + 16, 16)] for r in range(_S)]
                for jj in range(16):
                    i1 = base16 + jj
                    i2 = base16 + 16 + jj
                    acc = rv[robase + i1] * bv1[0][jj]
                    acc2 = rv[robase + i2] * bv2[0][jj]
                    for r in range(1, _S):
                        acc = acc + rv[robase + r * _CH + i1] * bv1[r][jj]
                        acc2 = acc2 + rv[robase + r * _CH + i2] * bv2[r][jj]
                    mv[mobase + i1] = acc
                    mv[mobase + i2] = acc2
                return carry

            lax.fori_loop(0, _CH // 32, group, 0)
        for u in range(2):
            pltpu.async_copy(mv.at[pl.ds(u * _CH, _CH)],
                             agg.at[dst_v[slot].at[u]],
                             sem_s[slot], add=True)

    load_in(0, 0)
    wait_in(0)
    fire_gathers(0)
    load_in(1, 1)

    def body(t, carry):
        first = t == 0
        wait_in(1)
        fire_gathers(1)
        wait_gathers(0)
        compute_and_scatter(0, first)
        load_in(0, 2 * t + 2)
        wait_gathers(1)
        compute_and_scatter(1, first)
        load_in(1, 2 * t + 3)
        wait_in(0)
        fire_gathers(0)
        return carry

    lax.fori_loop(0, _CPW // 4, body, 0)

    wait_in(1)
    wait_gathers(0)
    wait_scatter(0)
    wait_scatter(1)

    plsc.subcore_barrier()
    pltpu.sync_copy(agg.at[pl.ds(sid * _RPW, _RPW)],
                    p_hbm.at[cid, pl.ds(sid * _RPW, _RPW)])


def _wt(W):
    kt, ic, oc = W.shape
    Wp = jnp.pad(W, ((0, _KP - kt), (0, _OP - ic), (0, _OP - oc)))
    return Wp.transpose(1, 0, 2).reshape(_OP, 16, 128)


def _rootp(root):
    ic, oc = root.shape
    return jnp.pad(root, ((0, _OP - ic), (0, _OP - oc)))


def _bp(b):
    return jnp.pad(b, (0, _OP - b.shape[0])).reshape(1, _OP)


def kernel(x, edge_index, edge_attr,
           W1, root1, b1, W2, root2, b2, W3, root3, b3,
           W4, root4, b4, W5, root5, b5):
    eaT = jnp.pad(edge_attr.T, ((0, 0), (0, _EP - _E)))
    eip = jnp.pad(edge_index, ((0, 0), (0, _EP - _E)))
    fid, bas = _prep(eaT, eip)

    h, z = _mm1(x, _wt(W1))
    p = _sc_layer(z.reshape(_NG * 1024, _OP), fid, bas)[:, :_N, :]
    params = [(root1, b1, W2), (root2, b2, W3), (root3, b3, W4), (root4, b4, W5)]
    for (root, b, Wn) in params:
        h, z = _mm(p, h, _rootp(root), _bp(b), _wt(Wn))
        p = _sc_layer(z.reshape(_NG * 1024, _OP), fid, bas)[:, :_N, :]
    y = _fin(p, h, _rootp(root5), _bp(b5))
    return y.reshape(-1)


# speedup vs baseline: 1.3224x; 1.3224x over previous
import functools

import jax
import jax.numpy as jnp
from jax import lax
from jax.experimental import pallas as pl
from jax.experimental.pallas import tpu as pltpu
from jax.experimental.pallas import tpu_sc as plsc

_LOWER = -0.22703196
_UPPER = 0.36853024
_K = 5
_DIM = 3
_KT = 125
_S = 8
_N = 10000
_E = 160000
_EP = 163840
_OP = 16
_KP = 128
_CH = 128
_NCH = _EP // _CH
_NW = 32
_CPW = _NCH // _NW
_MB = 200
_EB = 1280
_NP = 10240
_RPW = _NP // 16
_NG = _N // 8


def _prep_body(ea_ref, ei_ref, fid_ref, bas_ref):
    i = pl.program_id(0)
    gid = i * _EB + lax.broadcasted_iota(jnp.int32, (1, _EB), 1)
    valid = (gid < _E).astype(jnp.float32)
    v = ea_ref[...] * (_K - 1.0)
    bot_f = jnp.floor(v)
    frac = v - bot_f
    bot = bot_f.astype(jnp.int32)
    src = ei_ref[0:1, :]
    dst = ei_ref[1:2, :]
    rows_b, rows_i = [], []
    for s in range(_S):
        w = jnp.ones((1, _EB), jnp.float32)
        idx = jnp.zeros((1, _EB), jnp.int32)
        stride = 1
        for d in range(_DIM):
            off = (s >> d) & 1
            fd = frac[d:d + 1, :]
            wd = fd if off else 1.0 - fd
            idd = jnp.clip(bot[d:d + 1, :] + off, 0, _K - 1)
            w = w * wd
            idx = idx + idd * stride
            stride *= _K
        rows_b.append(w * valid)
        fi = ((src >> 3) * 1024 + (idx >> 3) * 64
              + (src & 7) * 8 + (idx & 7))
        rows_i.append(fi)
    zpad = jnp.zeros((7, _EB), jnp.int32)
    rows16 = jnp.concatenate(rows_i + [dst, zpad], axis=0)
    rows8 = jnp.concatenate(rows_b, axis=0)
    for j in range(_EB // _CH):
        fid_ref[j] = rows16[:, j * _CH:(j + 1) * _CH]
        bas_ref[j] = rows8[:, j * _CH:(j + 1) * _CH]


_prep = pl.pallas_call(
    _prep_body,
    grid=(_EP // _EB,),
    in_specs=[
        pl.BlockSpec((_DIM, _EB), lambda i: (0, i)),
        pl.BlockSpec((2, _EB), lambda i: (0, i)),
    ],
    out_specs=[
        pl.BlockSpec((_EB // _CH, 16, _CH), lambda i: (i, 0, 0)),
        pl.BlockSpec((_EB // _CH, _S, _CH), lambda i: (i, 0, 0)),
    ],
    out_shape=[
        jax.ShapeDtypeStruct((_NCH, 16, _CH), jnp.int32),
        jax.ShapeDtypeStruct((_NCH, _S, _CH), jnp.float32),
    ],
)


def _elu(u):
    return jnp.where(u > 0.0, u, jnp.exp(u) - 1.0)


def _write_z(z_ref, h, wt_ref):
    for c in range(16):
        zc = jnp.dot(h, wt_ref[:, c, :], preferred_element_type=jnp.float32)
        z_ref[:, c, :, :] = zc.reshape(_MB // 8, 8, 128)


def _mm1_body(x_ref, wt_ref, h_ref, z_ref):
    t = (x_ref[...] - _LOWER) / (_UPPER - _LOWER) * 20.0 - 10.0
    t = jnp.clip(t, -10.0, 10.0)
    col0 = (lax.broadcasted_iota(jnp.int32, (1, _OP), 1) == 0)
    h = t * col0.astype(jnp.float32)
    h_ref[...] = h
    _write_z(z_ref, h, wt_ref)


_Z4 = jax.ShapeDtypeStruct((_NG, 16, 8, 128), jnp.float32)
_z4_spec = pl.BlockSpec((_MB // 8, 16, 8, 128), lambda i: (i, 0, 0, 0))
_wt_spec = pl.BlockSpec((16, _OP, 128), lambda i: (0, 0, 0))

_mm1 = pl.pallas_call(
    _mm1_body,
    grid=(_N // _MB,),
    in_specs=[
        pl.BlockSpec((_MB, 1), lambda i: (i, 0)),
        _wt_spec,
    ],
    out_specs=[
        pl.BlockSpec((_MB, _OP), lambda i: (i, 0)),
        _z4_spec,
    ],
    out_shape=[
        jax.ShapeDtypeStruct((_N, _OP), jnp.float32),
        _Z4,
    ],
)


def _mm_body(p_ref, hp_ref, root_ref, b_ref, wt_ref, h_ref, z_ref):
    agg = p_ref[0] + p_ref[1]
    r = jnp.dot(hp_ref[...], root_ref[...], preferred_element_type=jnp.float32)
    h = _elu(agg + r + b_ref[...])
    h_ref[...] = h
    _write_z(z_ref, h, wt_ref)


_mm = pl.pallas_call(
    _mm_body,
    grid=(_N // _MB,),
    in_specs=[
        pl.BlockSpec((2, _MB, _OP), lambda i: (0, i, 0)),
        pl.BlockSpec((_MB, _OP), lambda i: (i, 0)),
        pl.BlockSpec((_OP, _OP), lambda i: (0, 0)),
        pl.BlockSpec((1, _OP), lambda i: (0, 0)),
        _wt_spec,
    ],
    out_specs=[
        pl.BlockSpec((_MB, _OP), lambda i: (i, 0)),
        _z4_spec,
    ],
    out_shape=[
        jax.ShapeDtypeStruct((_N, _OP), jnp.float32),
        _Z4,
    ],
)


def _fin_body(p_ref, hp_ref, root_ref, b_ref, y_ref):
    agg = p_ref[0] + p_ref[1]
    r = jnp.dot(hp_ref[...], root_ref[...], preferred_element_type=jnp.float32)
    h = _elu(agg + r + b_ref[...])
    y_ref[...] = h[:, 0:1]


_fin = pl.pallas_call(
    _fin_body,
    grid=(_N // _MB,),
    in_specs=[
        pl.BlockSpec((2, _MB, _OP), lambda i: (0, i, 0)),
        pl.BlockSpec((_MB, _OP), lambda i: (i, 0)),
        pl.BlockSpec((_OP, _OP), lambda i: (0, 0)),
        pl.BlockSpec((1, _OP), lambda i: (0, 0)),
    ],
    out_specs=pl.BlockSpec((_MB, 1), lambda i: (i, 0)),
    out_shape=jax.ShapeDtypeStruct((_N, 1), jnp.float32),
)


_mesh = plsc.VectorSubcoreMesh(
    core_axis_name="c", subcore_axis_name="s", num_cores=2, num_subcores=16)


@functools.partial(
    pl.kernel,
    out_type=jax.ShapeDtypeStruct((2, _NP, _OP), jnp.float32),
    mesh=_mesh,
    compiler_params=pltpu.CompilerParams(use_tc_tiling_on_sc=False),
    scratch_types=[
        [pltpu.VMEM((2, 16, _CH), jnp.int32)] * 2,
        [pltpu.VMEM((2, _S, _CH), jnp.float32)] * 2,
        [pltpu.VMEM((2, _CH), jnp.int32)] * 2,
        [pltpu.VMEM((2 * _S * _CH, _OP), jnp.float32)] * 2,
        [pltpu.VMEM((2 * _CH, _OP), jnp.float32)] * 2,
        pltpu.VMEM((_CH, _OP), jnp.float32),
        pltpu.VMEM_SHARED((_NP, _OP), jnp.float32),
        [pltpu.SemaphoreType.DMA] * 2,
        [pltpu.SemaphoreType.DMA] * 2,
        [pltpu.SemaphoreType.DMA] * 2,
    ],
)
def _sc_layer(z_hbm, fid_hbm, bas_hbm, p_hbm,
              fid_v, bas_v, dst_v, rows_v, msg_v, zv, agg,
              sem_i, sem_g, sem_s):
    cid = lax.axis_index("c")
    sid = lax.axis_index("s")
    zero16 = jnp.zeros((16,), jnp.float32)
    for i in range(_CH):
        zv[i] = zero16
    for t in range(_RPW // _CH):
        pltpu.sync_copy(zv, agg.at[pl.ds(sid * _RPW + t * _CH, _CH)])
    plsc.subcore_barrier()

    wid = sid * 2 + cid
    base = wid * _CPW

    def load_in(slot, lch):
        ch = base + jnp.minimum(lch, _CPW // 2 - 1) * 2
        pltpu.async_copy(fid_hbm.at[pl.ds(ch, 2)], fid_v[slot], sem_i[slot])
        pltpu.async_copy(bas_hbm.at[pl.ds(ch, 2)], bas_v[slot], sem_i[slot])

    def wait_in(slot):
        pltpu.make_async_copy(fid_hbm.at[pl.ds(0, 2)], fid_v[slot],
                              sem_i[slot]).wait()
        pltpu.make_async_copy(bas_hbm.at[pl.ds(0, 2)], bas_v[slot],
                              sem_i[slot]).wait()

    def fire_gathers(slot):
        for u in range(2):
            for r in range(_S):
                pltpu.async_copy(
                    z_hbm.at[fid_v[slot].at[u, r]],
                    rows_v[slot].at[pl.ds((u * _S + r) * _CH, _CH)],
                    sem_g[slot])

    def wait_gathers(slot):
        for u in range(2):
            for r in range(_S):
                pltpu.make_async_copy(
                    z_hbm.at[fid_v[slot].at[u, r]],
                    rows_v[slot].at[pl.ds((u * _S + r) * _CH, _CH)],
                    sem_g[slot]).wait()

    def wait_scatter(slot):
        for u in range(2):
            pltpu.make_async_copy(msg_v[slot].at[pl.ds(u * _CH, _CH)],
                                  agg.at[dst_v[slot].at[u]],
                                  sem_s[slot]).wait()

    def compute_and_scatter(slot, first):
        rv = rows_v[slot]
        bv = bas_v[slot]
        mv = msg_v[slot]

        @pl.when(jnp.logical_not(first))
        def _():
            wait_scatter(slot)
        for u in range(2):
            for q in range(_CH // 16):
                dst_v[slot][u, pl.ds(q * 16, 16)] = (
                    fid_v[slot][u, 8, pl.ds(q * 16, 16)])
        for u in range(2):
            robase = u * _S * _CH
            mobase = u * _CH

            def group(g, carry):
                base16 = g * 32
                bv1 = [bv[u, r, pl.ds(base16, 16)] for r in range(_S)]
                bv2 = [bv[u, r, pl.ds(base16 + 16, 16)] for r in range(_S)]
                for jj in range(16):
                    i1 = base16 + jj
                    i2 = base16 + 16 + jj
                    acc = rv[robase + i1] * bv1[0][jj]
                    acc2 = rv[robase + i2] * bv2[0][jj]
                    for r in range(1, _S):
                        acc = acc + rv[robase + r * _CH + i1] * bv1[r][jj]
                        acc2 = acc2 + rv[robase + r * _CH + i2] * bv2[r][jj]
                    mv[mobase + i1] = acc
                    mv[mobase + i2] = acc2
                return carry

            lax.fori_loop(0, _CH // 32, group, 0)
        for u in range(2):
            pltpu.async_copy(mv.at[pl.ds(u * _CH, _CH)],
                             agg.at[dst_v[slot].at[u]],
                             sem_s[slot], add=True)

    load_in(0, 0)
    wait_in(0)
    fire_gathers(0)
    load_in(1, 1)

    def body(t, carry):
        first = t == 0
        wait_in(1)
        fire_gathers(1)
        wait_gathers(0)
        compute_and_scatter(0, first)
        load_in(0, 2 * t + 2)
        wait_gathers(1)
        compute_and_scatter(1, first)
        load_in(1, 2 * t + 3)
        wait_in(0)
        fire_gathers(0)
        return carry

    lax.fori_loop(0, _CPW // 4, body, 0)

    wait_in(1)
    wait_gathers(0)
    wait_scatter(0)
    wait_scatter(1)

    plsc.subcore_barrier()
    pltpu.sync_copy(agg.at[pl.ds(sid * _RPW, _RPW)],
                    p_hbm.at[cid, pl.ds(sid * _RPW, _RPW)])


def _wt(W):
    kt, ic, oc = W.shape
    Wp = jnp.pad(W, ((0, _KP - kt), (0, _OP - ic), (0, _OP - oc)))
    return Wp.transpose(1, 0, 2).reshape(_OP, 16, 128)


def _rootp(root):
    ic, oc = root.shape
    return jnp.pad(root, ((0, _OP - ic), (0, _OP - oc)))


def _bp(b):
    return jnp.pad(b, (0, _OP - b.shape[0])).reshape(1, _OP)


def kernel(x, edge_index, edge_attr,
           W1, root1, b1, W2, root2, b2, W3, root3, b3,
           W4, root4, b4, W5, root5, b5):
    eaT = jnp.pad(edge_attr.T, ((0, 0), (0, _EP - _E)))
    eip = jnp.pad(edge_index, ((0, 0), (0, _EP - _E)))
    fid, bas = _prep(eaT, eip)

    h, z = _mm1(x, _wt(W1))
    p = _sc_layer(z.reshape(_NG * 1024, _OP), fid, bas)[:, :_N, :]
    params = [(root1, b1, W2), (root2, b2, W3), (root3, b3, W4), (root4, b4, W5)]
    for (root, b, Wn) in params:
        h, z = _mm(p, h, _rootp(root), _bp(b), _wt(Wn))
        p = _sc_layer(z.reshape(_NG * 1024, _OP), fid, bas)[:, :_N, :]
    y = _fin(p, h, _rootp(root5), _bp(b5))
    return y.reshape(-1)
